# baseline (speedup 1.0000x reference)
import jax
import jax.numpy as jnp
from jax import lax
from jax.experimental import pallas as pl
from jax.experimental.pallas import tpu as pltpu
from jax.experimental.pallas import tpu_sc as plsc

CTX = 5
D = 50
DP = 56
NC, NS = 2, 16
NW = NC * NS
CHUNK = 128


def _sc_gather_sum(idx_v_hbm, idx_p_hbm, idx_s_hbm, emb_hbm, pemb_hbm,
                   semb_hbm, m_hbm, out_hbm,
                   iv, ip, isf, g1, g2, g3, acc, mv, sem1, sem2, sem3):
    n_chunks = out_hbm.shape[0] // (NW * CHUNK)
    wid = lax.axis_index("s") * NC + lax.axis_index("c")

    pltpu.sync_copy(m_hbm, mv)
    m = mv[...]

    def chunk_body(j, carry):
        row = wid * n_chunks + j
        pltpu.sync_copy(idx_v_hbm.at[row], iv)
        pltpu.sync_copy(idx_p_hbm.at[row], ip)
        pltpu.sync_copy(idx_s_hbm.at[row], isf)
        cp1 = pltpu.async_copy(emb_hbm.at[iv], g1, sem1)
        cp2 = pltpu.async_copy(pemb_hbm.at[ip], g2, sem2)
        cp3 = pltpu.async_copy(semb_hbm.at[isf], g3, sem3)
        cp1.wait()
        cp2.wait()
        cp3.wait()

        def row_body(r, c):
            for o in (0, 16, 32, 40):
                sl = pl.ds(o, 16)
                acc[r, sl] = g1[r, sl] + m * (g2[r, sl] + g3[r, sl])
            return c

        lax.fori_loop(0, CHUNK, row_body, 0, unroll=2)
        pltpu.sync_copy(acc, out_hbm.at[pl.ds(row * CHUNK, CHUNK)])
        return carry

    lax.fori_loop(0, n_chunks, chunk_body, 0)


def _mlp_body(x_ref, w1t_ref, b1_ref, w2t_ref, b2_ref, o_ref):
    x = x_ref[...]
    h = jnp.dot(x, w1t_ref[...], preferred_element_type=jnp.float32)
    h = jnp.maximum(h + b1_ref[...], 0.0)
    o = jnp.dot(h, w2t_ref[...], preferred_element_type=jnp.float32)
    o = o + b2_ref[...]
    mx = jnp.max(o, axis=1, keepdims=True)
    lse = jnp.log(jnp.sum(jnp.exp(o - mx), axis=1, keepdims=True)) + mx
    o_ref[...] = o - lse


def kernel(inputs, sub_words, p_inputs, s_inputs, emb, prefix_emb, suffix_emb,
           W1, b1, W2, b2):
    B = inputs.shape[1]
    R = B * CTX
    n_rows = R // CHUNK

    idx_v = inputs.T.reshape(n_rows, CHUNK).astype(jnp.int32)
    idx_p = p_inputs.T.reshape(n_rows, CHUNK).astype(jnp.int32)
    idx_s = s_inputs.T.reshape(n_rows, CHUNK).astype(jnp.int32)
    m_arr = jnp.broadcast_to(
        jnp.where(jnp.asarray(sub_words) != 0, 1.0, 0.0).astype(jnp.float32),
        (16,))

    pad = ((0, 0), (0, DP - D))
    emb56 = jnp.pad(emb, pad)
    pemb56 = jnp.pad(prefix_emb, pad)
    semb56 = jnp.pad(suffix_emb, pad)

    mesh = plsc.VectorSubcoreMesh(core_axis_name="c", subcore_axis_name="s")
    gathered = pl.kernel(
        _sc_gather_sum,
        out_type=jax.ShapeDtypeStruct((R, DP), jnp.float32),
        mesh=mesh,
        compiler_params=pltpu.CompilerParams(use_tc_tiling_on_sc=False),
        scratch_types=[
            pltpu.VMEM((CHUNK,), jnp.int32),
            pltpu.VMEM((CHUNK,), jnp.int32),
            pltpu.VMEM((CHUNK,), jnp.int32),
            pltpu.VMEM((CHUNK, DP), jnp.float32),
            pltpu.VMEM((CHUNK, DP), jnp.float32),
            pltpu.VMEM((CHUNK, DP), jnp.float32),
            pltpu.VMEM((CHUNK, DP), jnp.float32),
            pltpu.VMEM((16,), jnp.float32),
            pltpu.SemaphoreType.DMA,
            pltpu.SemaphoreType.DMA,
            pltpu.SemaphoreType.DMA,
        ],
    )(idx_v, idx_p, idx_s, emb56, pemb56, semb56, m_arr)

    x = gathered[:, :D].reshape(B, CTX * D)

    blk = 2048
    log_probs = pl.pallas_call(
        _mlp_body,
        out_shape=jax.ShapeDtypeStruct((B, W2.shape[0]), jnp.float32),
        grid=(B // blk,),
        in_specs=[
            pl.BlockSpec((blk, CTX * D), lambda i: (i, 0)),
            pl.BlockSpec((CTX * D, 128), lambda i: (0, 0)),
            pl.BlockSpec((1, 128), lambda i: (0, 0)),
            pl.BlockSpec((128, W2.shape[0]), lambda i: (0, 0)),
            pl.BlockSpec((1, W2.shape[0]), lambda i: (0, 0)),
        ],
        out_specs=pl.BlockSpec((blk, W2.shape[0]), lambda i: (i, 0)),
    )(x, W1.T, b1.reshape(1, -1), W2.T, b2.reshape(1, -1))

    return log_probs


# speedup vs baseline: 1.1243x; 1.1243x over previous
import jax
import jax.numpy as jnp
from jax import lax
from jax.experimental import pallas as pl
from jax.experimental.pallas import tpu as pltpu
from jax.experimental.pallas import tpu_sc as plsc

CTX = 5
D = 50
DP = 56
NC, NS = 2, 16
NW = NC * NS
CHUNK = 128


def _sc_gather_sum(idx_v_hbm, idx_p_hbm, idx_s_hbm, emb_hbm, pemb_hbm,
                   semb_hbm, m_hbm, out_hbm,
                   iv, ip, isf, g1, g2, g3, acc, mv, sem1, sem2, sem3):
    n_chunks = out_hbm.shape[0] // (NW * CHUNK)
    wid = lax.axis_index("s") * NC + lax.axis_index("c")

    pltpu.sync_copy(m_hbm, mv)
    m = mv[...]

    def chunk_body(j, carry):
        row = wid * n_chunks + j
        pltpu.sync_copy(idx_v_hbm.at[row], iv)
        pltpu.sync_copy(idx_p_hbm.at[row], ip)
        pltpu.sync_copy(idx_s_hbm.at[row], isf)
        cp1 = pltpu.async_copy(emb_hbm.at[iv], g1, sem1)
        cp2 = pltpu.async_copy(pemb_hbm.at[ip], g2, sem2)
        cp3 = pltpu.async_copy(semb_hbm.at[isf], g3, sem3)
        cp1.wait()
        cp2.wait()
        cp3.wait()

        def row_body(r, c):
            for o in (0, 16, 32, 40):
                sl = pl.ds(o, 16)
                acc[r, sl] = g1[r, sl] + m * (g2[r, sl] + g3[r, sl])
            return c

        lax.fori_loop(0, CHUNK, row_body, 0, unroll=2)
        pltpu.sync_copy(acc, out_hbm.at[pl.ds(row * CHUNK, CHUNK)])
        return carry

    lax.fori_loop(0, n_chunks, chunk_body, 0)


def _transpose_pad_body(xt_ref, o_ref):
    x = xt_ref[...]
    o_ref[...] = jnp.pad(x.T, ((0, 0), (0, DP - D)))


def _transpose_pad(table, vb=1024):
    V = table.shape[0]
    nblk = (V + vb - 1) // vb
    return pl.pallas_call(
        _transpose_pad_body,
        out_shape=jax.ShapeDtypeStruct((nblk * vb, DP), jnp.float32),
        grid=(nblk,),
        in_specs=[pl.BlockSpec((D, vb), lambda i: (0, i))],
        out_specs=pl.BlockSpec((vb, DP), lambda i: (i, 0)),
    )(table.T)


def _mlp_body(x_ref, w1t_ref, b1_ref, w2t_ref, b2_ref, o_ref):
    x = x_ref[...]
    h = jnp.dot(x, w1t_ref[...], preferred_element_type=jnp.float32)
    h = jnp.maximum(h + b1_ref[...], 0.0)
    o = jnp.dot(h, w2t_ref[...], preferred_element_type=jnp.float32)
    o = o + b2_ref[...]
    mx = jnp.max(o, axis=1, keepdims=True)
    lse = jnp.log(jnp.sum(jnp.exp(o - mx), axis=1, keepdims=True)) + mx
    o_ref[...] = o - lse


def kernel(inputs, sub_words, p_inputs, s_inputs, emb, prefix_emb, suffix_emb,
           W1, b1, W2, b2):
    B = inputs.shape[1]
    R = B * CTX
    n_rows = R // CHUNK

    idx_v = inputs.T.reshape(n_rows, CHUNK).astype(jnp.int32)
    idx_p = p_inputs.T.reshape(n_rows, CHUNK).astype(jnp.int32)
    idx_s = s_inputs.T.reshape(n_rows, CHUNK).astype(jnp.int32)
    m_arr = jnp.broadcast_to(
        jnp.where(jnp.asarray(sub_words) != 0, 1.0, 0.0).astype(jnp.float32),
        (16,))

    emb56 = _transpose_pad(emb)
    pemb56 = _transpose_pad(prefix_emb)
    semb56 = _transpose_pad(suffix_emb)

    mesh = plsc.VectorSubcoreMesh(core_axis_name="c", subcore_axis_name="s")
    gathered = pl.kernel(
        _sc_gather_sum,
        out_type=jax.ShapeDtypeStruct((R, DP), jnp.float32),
        mesh=mesh,
        compiler_params=pltpu.CompilerParams(use_tc_tiling_on_sc=False),
        scratch_types=[
            pltpu.VMEM((CHUNK,), jnp.int32),
            pltpu.VMEM((CHUNK,), jnp.int32),
            pltpu.VMEM((CHUNK,), jnp.int32),
            pltpu.VMEM((CHUNK, DP), jnp.float32),
            pltpu.VMEM((CHUNK, DP), jnp.float32),
            pltpu.VMEM((CHUNK, DP), jnp.float32),
            pltpu.VMEM((CHUNK, DP), jnp.float32),
            pltpu.VMEM((16,), jnp.float32),
            pltpu.SemaphoreType.DMA,
            pltpu.SemaphoreType.DMA,
            pltpu.SemaphoreType.DMA,
        ],
    )(idx_v, idx_p, idx_s, emb56, pemb56, semb56, m_arr)

    x = gathered[:, :D].reshape(B, CTX * D)

    blk = 2048
    log_probs = pl.pallas_call(
        _mlp_body,
        out_shape=jax.ShapeDtypeStruct((B, W2.shape[0]), jnp.float32),
        grid=(B // blk,),
        in_specs=[
            pl.BlockSpec((blk, CTX * D), lambda i: (i, 0)),
            pl.BlockSpec((CTX * D, 128), lambda i: (0, 0)),
            pl.BlockSpec((1, 128), lambda i: (0, 0)),
            pl.BlockSpec((128, W2.shape[0]), lambda i: (0, 0)),
            pl.BlockSpec((1, W2.shape[0]), lambda i: (0, 0)),
        ],
        out_specs=pl.BlockSpec((blk, W2.shape[0]), lambda i: (i, 0)),
    )(x, W1.T, b1.reshape(1, -1), W2.T, b2.reshape(1, -1))

    return log_probs


# speedup vs baseline: 2.3403x; 2.0816x over previous
import jax
import jax.numpy as jnp
from jax import lax
from jax.experimental import pallas as pl
from jax.experimental.pallas import tpu as pltpu
from jax.experimental.pallas import tpu_sc as plsc

CTX = 5
D = 50
W = 128
NC, NS = 2, 16
NW = NC * NS
CHUNK = 128


def _transpose_pad_body(xt_ref, o_ref):
    x = xt_ref[...]
    o_ref[...] = jnp.pad(x.T, ((0, 0), (0, W - D)))


def _transpose_pad(table, vb=2048):
    V = table.shape[0]
    nblk = (V + vb - 1) // vb
    return pl.pallas_call(
        _transpose_pad_body,
        out_shape=jax.ShapeDtypeStruct((nblk * vb, W), jnp.float32),
        grid=(nblk,),
        in_specs=[pl.BlockSpec((D, vb), lambda i: (0, i))],
        out_specs=pl.BlockSpec((vb, W), lambda i: (i, 0)),
    )(table.T)


def _sc_gather_sum(idx_v_hbm, idx_p_hbm, idx_s_hbm, emb_hbm, pemb_hbm,
                   semb_hbm, m_hbm, out_hbm,
                   iv, ip, isf, g1, g2, g3, acc, mv, sem1, sem2, sem3):
    n_chunks = out_hbm.shape[0] // (NW * CHUNK)
    wid = lax.axis_index("s") * NC + lax.axis_index("c")

    pltpu.sync_copy(m_hbm, mv)
    m = mv[...]

    def chunk_body(j, carry):
        row = wid * n_chunks + j
        pltpu.sync_copy(idx_v_hbm.at[row], iv)
        pltpu.sync_copy(idx_p_hbm.at[row], ip)
        pltpu.sync_copy(idx_s_hbm.at[row], isf)
        cp1 = pltpu.async_copy(emb_hbm.at[iv], g1, sem1)
        cp2 = pltpu.async_copy(pemb_hbm.at[ip], g2, sem2)
        cp3 = pltpu.async_copy(semb_hbm.at[isf], g3, sem3)
        cp1.wait()
        cp2.wait()
        cp3.wait()

        def row_body(r, c):
            for o in (0, 16, 32, 34):
                sl = pl.ds(o, 16)
                acc[r, sl] = g1[r, sl] + m * (g2[r, sl] + g3[r, sl])
            return c

        lax.fori_loop(0, CHUNK, row_body, 0, unroll=2)
        pltpu.sync_copy(acc, out_hbm.at[pl.ds(row * CHUNK, CHUNK)])
        return carry

    lax.fori_loop(0, n_chunks, chunk_body, 0)


def _mlp_body(x_ref, w1t_ref, b1_ref, w2t_ref, b2_ref, o_ref):
    x = x_ref[...]
    h = jnp.dot(x, w1t_ref[...], preferred_element_type=jnp.float32)
    h = jnp.maximum(h + b1_ref[...], 0.0)
    o = jnp.dot(h, w2t_ref[...], preferred_element_type=jnp.float32)
    o = o + b2_ref[...]
    mx = jnp.max(o, axis=1, keepdims=True)
    lse = jnp.log(jnp.sum(jnp.exp(o - mx), axis=1, keepdims=True)) + mx
    o_ref[...] = o - lse


def kernel(inputs, sub_words, p_inputs, s_inputs, emb, prefix_emb, suffix_emb,
           W1, b1, W2, b2):
    B = inputs.shape[1]
    R = B * CTX
    n_rows = R // CHUNK

    idx_v = inputs.T.reshape(n_rows, CHUNK).astype(jnp.int32)
    idx_p = p_inputs.T.reshape(n_rows, CHUNK).astype(jnp.int32)
    idx_s = s_inputs.T.reshape(n_rows, CHUNK).astype(jnp.int32)
    m_arr = jnp.broadcast_to(
        jnp.where(jnp.asarray(sub_words) != 0, 1.0, 0.0).astype(jnp.float32),
        (16,))

    emb128 = _transpose_pad(emb)
    pemb128 = _transpose_pad(prefix_emb)
    semb128 = _transpose_pad(suffix_emb)

    mesh = plsc.VectorSubcoreMesh(core_axis_name="c", subcore_axis_name="s")
    gathered = pl.kernel(
        _sc_gather_sum,
        out_type=jax.ShapeDtypeStruct((R, W), jnp.float32),
        mesh=mesh,
        scratch_types=[
            pltpu.VMEM((CHUNK,), jnp.int32),
            pltpu.VMEM((CHUNK,), jnp.int32),
            pltpu.VMEM((CHUNK,), jnp.int32),
            pltpu.VMEM((CHUNK, W), jnp.float32),
            pltpu.VMEM((CHUNK, W), jnp.float32),
            pltpu.VMEM((CHUNK, W), jnp.float32),
            pltpu.VMEM((CHUNK, W), jnp.float32),
            pltpu.VMEM((16,), jnp.float32),
            pltpu.SemaphoreType.DMA,
            pltpu.SemaphoreType.DMA,
            pltpu.SemaphoreType.DMA,
        ],
    )(idx_v, idx_p, idx_s, emb128, pemb128, semb128, m_arr)

    x = gathered[:, :D].reshape(B, CTX * D)

    blk = 2048
    log_probs = pl.pallas_call(
        _mlp_body,
        out_shape=jax.ShapeDtypeStruct((B, W2.shape[0]), jnp.float32),
        grid=(B // blk,),
        in_specs=[
            pl.BlockSpec((blk, CTX * D), lambda i: (i, 0)),
            pl.BlockSpec((CTX * D, 128), lambda i: (0, 0)),
            pl.BlockSpec((1, 128), lambda i: (0, 0)),
            pl.BlockSpec((128, W2.shape[0]), lambda i: (0, 0)),
            pl.BlockSpec((1, W2.shape[0]), lambda i: (0, 0)),
        ],
        out_specs=pl.BlockSpec((blk, W2.shape[0]), lambda i: (i, 0)),
    )(x, W1.T, b1.reshape(1, -1), W2.T, b2.reshape(1, -1))

    return log_probs


# speedup vs baseline: 2.4709x; 1.0558x over previous
import jax
import jax.numpy as jnp
from jax import lax
from jax.experimental import pallas as pl
from jax.experimental.pallas import tpu as pltpu
from jax.experimental.pallas import tpu_sc as plsc

CTX = 5
D = 50
W = 128
NC, NS = 2, 16
NW = NC * NS
CHUNK = 128


def _transpose_pad_body(xt_ref, o_ref):
    x = xt_ref[...]
    o_ref[...] = jnp.pad(x.T, ((0, 0), (0, W - D)))


def _transpose_pad(table, vb=2048):
    V = table.shape[0]
    nblk = (V + vb - 1) // vb
    return pl.pallas_call(
        _transpose_pad_body,
        out_shape=jax.ShapeDtypeStruct((nblk * vb, W), jnp.float32),
        grid=(nblk,),
        in_specs=[pl.BlockSpec((D, vb), lambda i: (0, i))],
        out_specs=pl.BlockSpec((vb, W), lambda i: (i, 0)),
    )(table.T)


def _sc_ps_gather(idx_p_hbm, idx_s_hbm, pemb_hbm, semb_hbm, m_hbm, psum_hbm,
                  ip, isf, g2, g3, acc, mv, sem2, sem3):
    n_chunks = psum_hbm.shape[0] // (NW * CHUNK)
    wid = lax.axis_index("s") * NC + lax.axis_index("c")

    pltpu.sync_copy(m_hbm, mv)
    m = mv[...]

    def chunk_body(j, carry):
        row = wid * n_chunks + j
        pltpu.sync_copy(idx_p_hbm.at[row], ip)
        pltpu.sync_copy(idx_s_hbm.at[row], isf)
        cp2 = pltpu.async_copy(pemb_hbm.at[ip], g2, sem2)
        cp3 = pltpu.async_copy(semb_hbm.at[isf], g3, sem3)
        cp2.wait()
        cp3.wait()

        def row_body(r, c):
            for o in (0, 16, 32, 34):
                sl = pl.ds(o, 16)
                acc[r, sl] = m * (g2[r, sl] + g3[r, sl])
            return c

        lax.fori_loop(0, CHUNK, row_body, 0, unroll=2)
        pltpu.sync_copy(acc, psum_hbm.at[pl.ds(row * CHUNK, CHUNK)])
        return carry

    lax.fori_loop(0, n_chunks, chunk_body, 0)


def _sc_emb_add(idx_v_hbm, emb_hbm, psum_hbm, out_hbm,
                iv, g1, pbuf, acc, sem1, semp):
    n_chunks = out_hbm.shape[0] // (NW * CHUNK)
    wid = lax.axis_index("s") * NC + lax.axis_index("c")

    def chunk_body(j, carry):
        row = wid * n_chunks + j
        pltpu.sync_copy(idx_v_hbm.at[row], iv)
        cpp = pltpu.async_copy(psum_hbm.at[pl.ds(row * CHUNK, CHUNK)], pbuf,
                               semp)
        cp1 = pltpu.async_copy(emb_hbm.at[iv], g1, sem1)
        cpp.wait()
        cp1.wait()

        def row_body(r, c):
            for o in (0, 16, 32, 34):
                sl = pl.ds(o, 16)
                acc[r, sl] = pbuf[r, sl] + g1[r, sl]
            return c

        lax.fori_loop(0, CHUNK, row_body, 0, unroll=2)
        pltpu.sync_copy(acc, out_hbm.at[pl.ds(row * CHUNK, CHUNK)])
        return carry

    lax.fori_loop(0, n_chunks, chunk_body, 0)


def _mlp_body(x_ref, w1t_ref, b1_ref, w2t_ref, b2_ref, o_ref):
    x = x_ref[...]
    h = jnp.dot(x, w1t_ref[...], preferred_element_type=jnp.float32)
    h = jnp.maximum(h + b1_ref[...], 0.0)
    o = jnp.dot(h, w2t_ref[...], preferred_element_type=jnp.float32)
    o = o + b2_ref[...]
    mx = jnp.max(o, axis=1, keepdims=True)
    lse = jnp.log(jnp.sum(jnp.exp(o - mx), axis=1, keepdims=True)) + mx
    o_ref[...] = o - lse


def kernel(inputs, sub_words, p_inputs, s_inputs, emb, prefix_emb, suffix_emb,
           W1, b1, W2, b2):
    B = inputs.shape[1]
    R = B * CTX
    n_rows = R // CHUNK

    idx_v = inputs.T.reshape(n_rows, CHUNK).astype(jnp.int32)
    idx_p = p_inputs.T.reshape(n_rows, CHUNK).astype(jnp.int32)
    idx_s = s_inputs.T.reshape(n_rows, CHUNK).astype(jnp.int32)
    m_arr = jnp.broadcast_to(
        jnp.where(jnp.asarray(sub_words) != 0, 1.0, 0.0).astype(jnp.float32),
        (16,))

    pemb128 = _transpose_pad(prefix_emb)
    semb128 = _transpose_pad(suffix_emb)

    mesh = plsc.VectorSubcoreMesh(core_axis_name="c", subcore_axis_name="s")
    psum = pl.kernel(
        _sc_ps_gather,
        out_type=jax.ShapeDtypeStruct((R, W), jnp.float32),
        mesh=mesh,
        scratch_types=[
            pltpu.VMEM((CHUNK,), jnp.int32),
            pltpu.VMEM((CHUNK,), jnp.int32),
            pltpu.VMEM((CHUNK, W), jnp.float32),
            pltpu.VMEM((CHUNK, W), jnp.float32),
            pltpu.VMEM((CHUNK, W), jnp.float32),
            pltpu.VMEM((16,), jnp.float32),
            pltpu.SemaphoreType.DMA,
            pltpu.SemaphoreType.DMA,
        ],
    )(idx_p, idx_s, pemb128, semb128, m_arr)

    emb128 = _transpose_pad(emb)

    gathered = pl.kernel(
        _sc_emb_add,
        out_type=jax.ShapeDtypeStruct((R, W), jnp.float32),
        mesh=mesh,
        scratch_types=[
            pltpu.VMEM((CHUNK,), jnp.int32),
            pltpu.VMEM((CHUNK, W), jnp.float32),
            pltpu.VMEM((CHUNK, W), jnp.float32),
            pltpu.VMEM((CHUNK, W), jnp.float32),
            pltpu.SemaphoreType.DMA,
            pltpu.SemaphoreType.DMA,
        ],
    )(idx_v, emb128, psum)

    x = gathered[:, :D].reshape(B, CTX * D)

    blk = 2048
    log_probs = pl.pallas_call(
        _mlp_body,
        out_shape=jax.ShapeDtypeStruct((B, W2.shape[0]), jnp.float32),
        grid=(B // blk,),
        in_specs=[
            pl.BlockSpec((blk, CTX * D), lambda i: (i, 0)),
            pl.BlockSpec((CTX * D, 128), lambda i: (0, 0)),
            pl.BlockSpec((1, 128), lambda i: (0, 0)),
            pl.BlockSpec((128, W2.shape[0]), lambda i: (0, 0)),
            pl.BlockSpec((1, W2.shape[0]), lambda i: (0, 0)),
        ],
        out_specs=pl.BlockSpec((blk, W2.shape[0]), lambda i: (i, 0)),
    )(x, W1.T, b1.reshape(1, -1), W2.T, b2.reshape(1, -1))

    return log_probs


# speedup vs baseline: 3.0202x; 1.2223x over previous
import jax
import jax.numpy as jnp
from jax import lax
from jax.experimental import pallas as pl
from jax.experimental.pallas import tpu as pltpu
from jax.experimental.pallas import tpu_sc as plsc

CTX = 5
D = 50
W = 128
NC, NS = 2, 16
NW = NC * NS
CHUNK = 128


def _transpose_pad_body(xt_ref, o_ref):
    x = xt_ref[...]
    o_ref[...] = jnp.pad(x.T, ((0, 0), (0, W - D)))


def _transpose_pad(table, vb=4096):
    V = table.shape[0]
    nblk = (V + vb - 1) // vb
    return pl.pallas_call(
        _transpose_pad_body,
        out_shape=jax.ShapeDtypeStruct((nblk * vb, W), jnp.float32),
        grid=(nblk,),
        in_specs=[pl.BlockSpec((D, vb), lambda i: (0, i))],
        out_specs=pl.BlockSpec((vb, W), lambda i: (i, 0)),
    )(table.T)


def _sc_ps_gather(idx_p_hbm, idx_s_hbm, pemb_hbm, semb_hbm, m_hbm, psum_hbm,
                  ip, isf, g2, g3, acc, mv, sem2, sem3):
    n_chunks = psum_hbm.shape[0] // (NW * CHUNK)
    wid = lax.axis_index("s") * NC + lax.axis_index("c")

    pltpu.sync_copy(m_hbm, mv)
    m = mv[...]

    def chunk_body(j, carry):
        row = wid * n_chunks + j
        pltpu.sync_copy(idx_p_hbm.at[row], ip)
        pltpu.sync_copy(idx_s_hbm.at[row], isf)
        cp2 = pltpu.async_copy(pemb_hbm.at[ip], g2, sem2)
        cp3 = pltpu.async_copy(semb_hbm.at[isf], g3, sem3)
        cp2.wait()
        cp3.wait()

        def row_body(r, c):
            for o in (0, 16, 32, 34):
                sl = pl.ds(o, 16)
                acc[r, sl] = m * (g2[r, sl] + g3[r, sl])
            return c

        lax.fori_loop(0, CHUNK, row_body, 0, unroll=2)
        pltpu.sync_copy(acc, psum_hbm.at[pl.ds(row * CHUNK, CHUNK)])
        return carry

    lax.fori_loop(0, n_chunks, chunk_body, 0)


def _sc_emb_add(idx_v_hbm, emb_hbm, psum_hbm, out_hbm,
                iv, g1, pbuf, acc, sem1, semp):
    n_chunks = out_hbm.shape[0] // (NW * CHUNK)
    wid = lax.axis_index("s") * NC + lax.axis_index("c")

    def chunk_body(j, carry):
        row = wid * n_chunks + j
        pltpu.sync_copy(idx_v_hbm.at[row], iv)
        cpp = pltpu.async_copy(psum_hbm.at[pl.ds(row * CHUNK, CHUNK)], pbuf,
                               semp)
        cp1 = pltpu.async_copy(emb_hbm.at[iv], g1, sem1)
        cpp.wait()
        cp1.wait()

        def row_body(r, c):
            for o in (0, 16, 32, 34):
                sl = pl.ds(o, 16)
                acc[r, sl] = pbuf[r, sl] + g1[r, sl]
            return c

        lax.fori_loop(0, CHUNK, row_body, 0, unroll=2)
        pltpu.sync_copy(acc, out_hbm.at[pl.ds(row * CHUNK, CHUNK)])
        return carry

    lax.fori_loop(0, n_chunks, chunk_body, 0)


def _mlp_body(x_ref, w1t_ref, b1_ref, w2t_ref, b2_ref, o_ref):
    x = x_ref[...]
    h = jnp.dot(x, w1t_ref[...], preferred_element_type=jnp.float32)
    h = jnp.maximum(h + b1_ref[...], 0.0)
    o = jnp.dot(h, w2t_ref[...], preferred_element_type=jnp.float32)
    o = o + b2_ref[...]
    mx = jnp.max(o, axis=1, keepdims=True)
    lse = jnp.log(jnp.sum(jnp.exp(o - mx), axis=1, keepdims=True)) + mx
    o_ref[...] = o - lse


def kernel(inputs, sub_words, p_inputs, s_inputs, emb, prefix_emb, suffix_emb,
           W1, b1, W2, b2):
    B = inputs.shape[1]
    R = B * CTX
    n_rows = R // CHUNK

    idx_v = inputs.T.reshape(n_rows, CHUNK).astype(jnp.int32)
    idx_p = p_inputs.T.reshape(n_rows, CHUNK).astype(jnp.int32)
    idx_s = s_inputs.T.reshape(n_rows, CHUNK).astype(jnp.int32)
    m_arr = jnp.broadcast_to(
        jnp.where(jnp.asarray(sub_words) != 0, 1.0, 0.0).astype(jnp.float32),
        (16,))

    pemb128 = _transpose_pad(prefix_emb)
    semb128 = _transpose_pad(suffix_emb)

    mesh = plsc.VectorSubcoreMesh(core_axis_name="c", subcore_axis_name="s")
    psum = pl.kernel(
        _sc_ps_gather,
        out_type=jax.ShapeDtypeStruct((R, W), jnp.float32),
        mesh=mesh,
        scratch_types=[
            pltpu.VMEM((CHUNK,), jnp.int32),
            pltpu.VMEM((CHUNK,), jnp.int32),
            pltpu.VMEM((CHUNK, W), jnp.float32),
            pltpu.VMEM((CHUNK, W), jnp.float32),
            pltpu.VMEM((CHUNK, W), jnp.float32),
            pltpu.VMEM((16,), jnp.float32),
            pltpu.SemaphoreType.DMA,
            pltpu.SemaphoreType.DMA,
        ],
    )(idx_p, idx_s, pemb128, semb128, m_arr)

    emb128 = _transpose_pad(emb)

    gathered = pl.kernel(
        _sc_emb_add,
        out_type=jax.ShapeDtypeStruct((R, W), jnp.float32),
        mesh=mesh,
        scratch_types=[
            pltpu.VMEM((CHUNK,), jnp.int32),
            pltpu.VMEM((CHUNK, W), jnp.float32),
            pltpu.VMEM((CHUNK, W), jnp.float32),
            pltpu.VMEM((CHUNK, W), jnp.float32),
            pltpu.SemaphoreType.DMA,
            pltpu.SemaphoreType.DMA,
        ],
    )(idx_v, emb128, psum)

    x = gathered[:, :D].reshape(B, CTX * D)

    blk = 2048
    log_probs = pl.pallas_call(
        _mlp_body,
        out_shape=jax.ShapeDtypeStruct((B, W2.shape[0]), jnp.float32),
        grid=(B // blk,),
        in_specs=[
            pl.BlockSpec((blk, CTX * D), lambda i: (i, 0)),
            pl.BlockSpec((CTX * D, 128), lambda i: (0, 0)),
            pl.BlockSpec((1, 128), lambda i: (0, 0)),
            pl.BlockSpec((128, W2.shape[0]), lambda i: (0, 0)),
            pl.BlockSpec((1, W2.shape[0]), lambda i: (0, 0)),
        ],
        out_specs=pl.BlockSpec((blk, W2.shape[0]), lambda i: (i, 0)),
    )(x, W1.T, b1.reshape(1, -1), W2.T, b2.reshape(1, -1))

    return log_probs


# speedup vs baseline: 3.3741x; 1.1172x over previous
import jax
import jax.numpy as jnp
from jax import lax
from jax.experimental import pallas as pl
from jax.experimental.pallas import tpu as pltpu
from jax.experimental.pallas import tpu_sc as plsc

CTX = 5
D = 50
W = 128
NC, NS = 2, 16
NW = NC * NS
CHUNK = 128


def _transpose_pad_body(xt_ref, o_ref):
    x = xt_ref[...]
    o_ref[...] = jnp.pad(x.T, ((0, 0), (0, W - D)))


def _transpose_pad(table, vb=8192):
    V = table.shape[0]
    nblk = (V + vb - 1) // vb
    return pl.pallas_call(
        _transpose_pad_body,
        out_shape=jax.ShapeDtypeStruct((nblk * vb, W), jnp.float32),
        grid=(nblk,),
        in_specs=[pl.BlockSpec((D, vb), lambda i: (0, i))],
        out_specs=pl.BlockSpec((vb, W), lambda i: (i, 0)),
    )(table.T)


def _sc_ps_gather(idx_p_hbm, idx_s_hbm, pemb_hbm, semb_hbm, m_hbm, psum_hbm,
                  ip, isf, g2, g3, acc, mv, sem2, sem3):
    n_chunks = psum_hbm.shape[0] // (NW * CHUNK)
    wid = lax.axis_index("s") * NC + lax.axis_index("c")

    pltpu.sync_copy(m_hbm, mv)
    m = mv[...]

    def chunk_body(j, carry):
        row = wid * n_chunks + j
        pltpu.sync_copy(idx_p_hbm.at[row], ip)
        pltpu.sync_copy(idx_s_hbm.at[row], isf)
        cp2 = pltpu.async_copy(pemb_hbm.at[ip], g2, sem2)
        cp3 = pltpu.async_copy(semb_hbm.at[isf], g3, sem3)
        cp2.wait()
        cp3.wait()

        def row_body(r, c):
            for o in (0, 16, 32, 34):
                sl = pl.ds(o, 16)
                acc[r, sl] = m * (g2[r, sl] + g3[r, sl])
            return c

        lax.fori_loop(0, CHUNK, row_body, 0, unroll=2)
        pltpu.sync_copy(acc, psum_hbm.at[pl.ds(row * CHUNK, CHUNK)])
        return carry

    lax.fori_loop(0, n_chunks, chunk_body, 0)


def _sc_emb_add(idx_v_hbm, emb_hbm, psum_hbm, out_hbm,
                iv, g1, pbuf, acc, sem1, semp):
    n_chunks = out_hbm.shape[0] // (NW * CHUNK)
    wid = lax.axis_index("s") * NC + lax.axis_index("c")

    def chunk_body(j, carry):
        row = wid * n_chunks + j
        pltpu.sync_copy(idx_v_hbm.at[row], iv)
        cpp = pltpu.async_copy(psum_hbm.at[pl.ds(row * CHUNK, CHUNK)], pbuf,
                               semp)
        cp1 = pltpu.async_copy(emb_hbm.at[iv], g1, sem1)
        cpp.wait()
        cp1.wait()

        def row_body(r, c):
            for o in (0, 16, 32, 34):
                sl = pl.ds(o, 16)
                acc[r, sl] = pbuf[r, sl] + g1[r, sl]
            return c

        lax.fori_loop(0, CHUNK, row_body, 0, unroll=2)
        pltpu.sync_copy(acc, out_hbm.at[pl.ds(row * CHUNK, CHUNK)])
        return carry

    lax.fori_loop(0, n_chunks, chunk_body, 0)


def _mlp_body(x_ref, w1t_ref, b1_ref, w2t_ref, b2_ref, o_ref):
    x = x_ref[...]
    h = jnp.dot(x, w1t_ref[...], preferred_element_type=jnp.float32)
    h = jnp.maximum(h + b1_ref[...], 0.0)
    o = jnp.dot(h, w2t_ref[...], preferred_element_type=jnp.float32)
    o = o + b2_ref[...]
    mx = jnp.max(o, axis=1, keepdims=True)
    lse = jnp.log(jnp.sum(jnp.exp(o - mx), axis=1, keepdims=True)) + mx
    o_ref[...] = o - lse


def kernel(inputs, sub_words, p_inputs, s_inputs, emb, prefix_emb, suffix_emb,
           W1, b1, W2, b2):
    B = inputs.shape[1]
    R = B * CTX
    n_rows = R // CHUNK

    idx_v = inputs.T.reshape(n_rows, CHUNK).astype(jnp.int32)
    idx_p = p_inputs.T.reshape(n_rows, CHUNK).astype(jnp.int32)
    idx_s = s_inputs.T.reshape(n_rows, CHUNK).astype(jnp.int32)
    m_arr = jnp.broadcast_to(
        jnp.where(jnp.asarray(sub_words) != 0, 1.0, 0.0).astype(jnp.float32),
        (16,))

    pemb128 = _transpose_pad(prefix_emb)
    semb128 = _transpose_pad(suffix_emb)

    mesh = plsc.VectorSubcoreMesh(core_axis_name="c", subcore_axis_name="s")
    psum = pl.kernel(
        _sc_ps_gather,
        out_type=jax.ShapeDtypeStruct((R, W), jnp.float32),
        mesh=mesh,
        scratch_types=[
            pltpu.VMEM((CHUNK,), jnp.int32),
            pltpu.VMEM((CHUNK,), jnp.int32),
            pltpu.VMEM((CHUNK, W), jnp.float32),
            pltpu.VMEM((CHUNK, W), jnp.float32),
            pltpu.VMEM((CHUNK, W), jnp.float32),
            pltpu.VMEM((16,), jnp.float32),
            pltpu.SemaphoreType.DMA,
            pltpu.SemaphoreType.DMA,
        ],
    )(idx_p, idx_s, pemb128, semb128, m_arr)

    emb128 = _transpose_pad(emb)

    gathered = pl.kernel(
        _sc_emb_add,
        out_type=jax.ShapeDtypeStruct((R, W), jnp.float32),
        mesh=mesh,
        scratch_types=[
            pltpu.VMEM((CHUNK,), jnp.int32),
            pltpu.VMEM((CHUNK, W), jnp.float32),
            pltpu.VMEM((CHUNK, W), jnp.float32),
            pltpu.VMEM((CHUNK, W), jnp.float32),
            pltpu.SemaphoreType.DMA,
            pltpu.SemaphoreType.DMA,
        ],
    )(idx_v, emb128, psum)

    x = gathered[:, :D].reshape(B, CTX * D)

    blk = 2048
    log_probs = pl.pallas_call(
        _mlp_body,
        out_shape=jax.ShapeDtypeStruct((B, W2.shape[0]), jnp.float32),
        grid=(B // blk,),
        in_specs=[
            pl.BlockSpec((blk, CTX * D), lambda i: (i, 0)),
            pl.BlockSpec((CTX * D, 128), lambda i: (0, 0)),
            pl.BlockSpec((1, 128), lambda i: (0, 0)),
            pl.BlockSpec((128, W2.shape[0]), lambda i: (0, 0)),
            pl.BlockSpec((1, W2.shape[0]), lambda i: (0, 0)),
        ],
        out_specs=pl.BlockSpec((blk, W2.shape[0]), lambda i: (i, 0)),
    )(x, W1.T, b1.reshape(1, -1), W2.T, b2.reshape(1, -1))

    return log_probs


# speedup vs baseline: 3.4617x; 1.0260x over previous
import jax
import jax.numpy as jnp
from jax import lax
from jax.experimental import pallas as pl
from jax.experimental.pallas import tpu as pltpu
from jax.experimental.pallas import tpu_sc as plsc

CTX = 5
D = 50
W = 128
NC, NS = 2, 16
NW = NC * NS
CHUNK = 128


def _transpose_pad_body(xt_ref, o_ref):
    x = xt_ref[...]
    o_ref[...] = jnp.pad(x.T, ((0, 0), (0, W - D)))


def _transpose_pad(table, vb=16384):
    V = table.shape[0]
    nblk = (V + vb - 1) // vb
    return pl.pallas_call(
        _transpose_pad_body,
        out_shape=jax.ShapeDtypeStruct((nblk * vb, W), jnp.float32),
        grid=(nblk,),
        in_specs=[pl.BlockSpec((D, vb), lambda i: (0, i))],
        out_specs=pl.BlockSpec((vb, W), lambda i: (i, 0)),
    )(table.T)


def _sc_ps_gather(idx_p_hbm, idx_s_hbm, pemb_hbm, semb_hbm, m_hbm, psum_hbm,
                  ip, isf, g2, g3, acc, mv, sem2, sem3):
    n_chunks = psum_hbm.shape[0] // (NW * CHUNK)
    wid = lax.axis_index("s") * NC + lax.axis_index("c")

    pltpu.sync_copy(m_hbm, mv)
    m = mv[...]

    def chunk_body(j, carry):
        row = wid * n_chunks + j
        pltpu.sync_copy(idx_p_hbm.at[row], ip)
        pltpu.sync_copy(idx_s_hbm.at[row], isf)
        cp2 = pltpu.async_copy(pemb_hbm.at[ip], g2, sem2)
        cp3 = pltpu.async_copy(semb_hbm.at[isf], g3, sem3)
        cp2.wait()
        cp3.wait()

        def row_body(r, c):
            for o in (0, 16, 32, 34):
                sl = pl.ds(o, 16)
                acc[r, sl] = m * (g2[r, sl] + g3[r, sl])
            return c

        lax.fori_loop(0, CHUNK, row_body, 0, unroll=2)
        pltpu.sync_copy(acc, psum_hbm.at[pl.ds(row * CHUNK, CHUNK)])
        return carry

    lax.fori_loop(0, n_chunks, chunk_body, 0)


def _sc_emb_add(idx_v_hbm, emb_hbm, psum_hbm, out_hbm,
                iv, g1, pbuf, acc, sem1, semp):
    n_chunks = out_hbm.shape[0] // (NW * CHUNK)
    wid = lax.axis_index("s") * NC + lax.axis_index("c")

    def chunk_body(j, carry):
        row = wid * n_chunks + j
        pltpu.sync_copy(idx_v_hbm.at[row], iv)
        cpp = pltpu.async_copy(psum_hbm.at[pl.ds(row * CHUNK, CHUNK)], pbuf,
                               semp)
        cp1 = pltpu.async_copy(emb_hbm.at[iv], g1, sem1)
        cpp.wait()
        cp1.wait()

        def row_body(r, c):
            for o in (0, 16, 32, 34):
                sl = pl.ds(o, 16)
                acc[r, sl] = pbuf[r, sl] + g1[r, sl]
            return c

        lax.fori_loop(0, CHUNK, row_body, 0, unroll=2)
        pltpu.sync_copy(acc, out_hbm.at[pl.ds(row * CHUNK, CHUNK)])
        return carry

    lax.fori_loop(0, n_chunks, chunk_body, 0)


def _mlp_body(x_ref, w1t_ref, b1_ref, w2t_ref, b2_ref, o_ref):
    x = x_ref[...]
    h = jnp.dot(x, w1t_ref[...], preferred_element_type=jnp.float32)
    h = jnp.maximum(h + b1_ref[...], 0.0)
    o = jnp.dot(h, w2t_ref[...], preferred_element_type=jnp.float32)
    o = o + b2_ref[...]
    mx = jnp.max(o, axis=1, keepdims=True)
    lse = jnp.log(jnp.sum(jnp.exp(o - mx), axis=1, keepdims=True)) + mx
    o_ref[...] = o - lse


def kernel(inputs, sub_words, p_inputs, s_inputs, emb, prefix_emb, suffix_emb,
           W1, b1, W2, b2):
    B = inputs.shape[1]
    R = B * CTX
    n_rows = R // CHUNK

    idx_v = inputs.T.reshape(n_rows, CHUNK).astype(jnp.int32)
    idx_p = p_inputs.T.reshape(n_rows, CHUNK).astype(jnp.int32)
    idx_s = s_inputs.T.reshape(n_rows, CHUNK).astype(jnp.int32)
    m_arr = jnp.broadcast_to(
        jnp.where(jnp.asarray(sub_words) != 0, 1.0, 0.0).astype(jnp.float32),
        (16,))

    pemb128 = _transpose_pad(prefix_emb)
    semb128 = _transpose_pad(suffix_emb)

    mesh = plsc.VectorSubcoreMesh(core_axis_name="c", subcore_axis_name="s")
    psum = pl.kernel(
        _sc_ps_gather,
        out_type=jax.ShapeDtypeStruct((R, W), jnp.float32),
        mesh=mesh,
        scratch_types=[
            pltpu.VMEM((CHUNK,), jnp.int32),
            pltpu.VMEM((CHUNK,), jnp.int32),
            pltpu.VMEM((CHUNK, W), jnp.float32),
            pltpu.VMEM((CHUNK, W), jnp.float32),
            pltpu.VMEM((CHUNK, W), jnp.float32),
            pltpu.VMEM((16,), jnp.float32),
            pltpu.SemaphoreType.DMA,
            pltpu.SemaphoreType.DMA,
        ],
    )(idx_p, idx_s, pemb128, semb128, m_arr)

    emb128 = _transpose_pad(emb)

    gathered = pl.kernel(
        _sc_emb_add,
        out_type=jax.ShapeDtypeStruct((R, W), jnp.float32),
        mesh=mesh,
        scratch_types=[
            pltpu.VMEM((CHUNK,), jnp.int32),
            pltpu.VMEM((CHUNK, W), jnp.float32),
            pltpu.VMEM((CHUNK, W), jnp.float32),
            pltpu.VMEM((CHUNK, W), jnp.float32),
            pltpu.SemaphoreType.DMA,
            pltpu.SemaphoreType.DMA,
        ],
    )(idx_v, emb128, psum)

    x = gathered[:, :D].reshape(B, CTX * D)

    blk = 2048
    log_probs = pl.pallas_call(
        _mlp_body,
        out_shape=jax.ShapeDtypeStruct((B, W2.shape[0]), jnp.float32),
        grid=(B // blk,),
        in_specs=[
            pl.BlockSpec((blk, CTX * D), lambda i: (i, 0)),
            pl.BlockSpec((CTX * D, 128), lambda i: (0, 0)),
            pl.BlockSpec((1, 128), lambda i: (0, 0)),
            pl.BlockSpec((128, W2.shape[0]), lambda i: (0, 0)),
            pl.BlockSpec((1, W2.shape[0]), lambda i: (0, 0)),
        ],
        out_specs=pl.BlockSpec((blk, W2.shape[0]), lambda i: (i, 0)),
    )(x, W1.T, b1.reshape(1, -1), W2.T, b2.reshape(1, -1))

    return log_probs
